# baseline (speedup 1.0000x reference)
import jax
import jax.numpy as jnp
from jax.experimental import pallas as pl
from jax.experimental.pallas import tpu as pltpu

_N = 2_000_000
_C = 17
_MAX_OUT = 5
_IOU_THR = 0.3
_SCORE_THR = 0.5
_IMG_SIZE = 128.0
_CLIP_MAX = 1e8

_NL = 128
_NT = 2_097_152
_NR = _NT // _NL
_BR = 2048
_G = _NR // _BR
_NEG = float("-inf")
_BIG_I = 2 ** 30


def _nms_body(cy_ref, cx_ref, h_ref, w_ref, s_ref, out_ref,
              xs1, ys1, xs2, ys2, sms, smf, smi, sel_i, sel_ok):
    k = pl.program_id(0)
    i = pl.program_id(1)

    rows2d = jax.lax.broadcasted_iota(jnp.int32, (_BR, _NL), 0)
    lanes2d = jax.lax.broadcasted_iota(jnp.int32, (_BR, _NL), 1)

    def update_running(sm):
        bm = jnp.max(sm)
        idxmat = (i * _BR + rows2d) * _NL + lanes2d
        cand = jnp.min(jnp.where(sm == bm, idxmat, _BIG_I))
        cur_m = jnp.where(i == 0, _NEG, smf[0])
        cur_i = jnp.where(i == 0, 0, smi[0])
        better = bm > cur_m
        smf[0] = jnp.where(better, bm, cur_m)
        smi[0] = jnp.where(better, cand, cur_i)

    @pl.when(k == 0)
    def _decode():
        cy = cy_ref[...]
        cx = cx_ref[...]
        h = h_ref[...]
        w = w_ref[...]
        s = s_ref[...]
        y1 = jnp.clip(cy - h * 0.5, 0.0, _CLIP_MAX)
        x1 = jnp.clip(cx - w * 0.5, 0.0, _CLIP_MAX)
        y2 = cy + h * 0.5
        x2 = cx + w * 0.5
        sm = jnp.where(s >= _SCORE_THR, s, _NEG)
        r0 = i * _BR
        xs1[pl.ds(r0, _BR), :] = x1
        ys1[pl.ds(r0, _BR), :] = y1
        xs2[pl.ds(r0, _BR), :] = x2
        ys2[pl.ds(r0, _BR), :] = y2
        sms[pl.ds(r0, _BR), :] = sm
        update_running(sm)

    @pl.when(k > 0)
    def _suppress():
        r0 = i * _BR
        x1 = xs1[pl.ds(r0, _BR), :]
        y1 = ys1[pl.ds(r0, _BR), :]
        x2 = xs2[pl.ds(r0, _BR), :]
        y2 = ys2[pl.ds(r0, _BR), :]
        sm = sms[pl.ds(r0, _BR), :]
        X1 = smf[1]
        Y1 = smf[2]
        X2 = smf[3]
        Y2 = smf[4]
        A = smf[5]
        p_idx = smi[1]
        iw = jnp.maximum(jnp.minimum(x2, X2) - jnp.maximum(x1, X1), 0.0)
        ih = jnp.maximum(jnp.minimum(y2, Y2) - jnp.maximum(y1, Y1), 0.0)
        inter = iw * ih
        areas = (x2 - x1) * (y2 - y1)
        iou = inter / (areas + A - inter + 1e-9)
        idxmat = (i * _BR + rows2d) * _NL + lanes2d
        kill = jnp.logical_or(iou > _IOU_THR, idxmat == p_idx)
        sm = jnp.where(kill, _NEG, sm)
        sms[pl.ds(r0, _BR), :] = sm
        update_running(sm)

    @pl.when(i == _G - 1)
    def _finalize():
        idx = smi[0]
        val = smf[0]
        sel_i[k] = idx
        sel_ok[k] = jnp.where(val > _NEG, 1, 0)
        r = idx // _NL
        l = idx % _NL
        lane1 = jax.lax.broadcasted_iota(jnp.int32, (1, _NL), 1)

        def pick(ref):
            row = ref[pl.ds(r, 1), :]
            return jnp.max(jnp.where(lane1 == l, row, _NEG))

        X1 = pick(xs1)
        Y1 = pick(ys1)
        X2 = pick(xs2)
        Y2 = pick(ys2)
        smf[1] = X1
        smf[2] = Y1
        smf[3] = X2
        smf[4] = Y2
        smf[5] = (X2 - X1) * (Y2 - Y1)
        smi[1] = idx

        @pl.when(k == _MAX_OUT - 1)
        def _emit():
            r8 = jax.lax.broadcasted_iota(jnp.int32, (8, _NL), 0)
            l8 = jax.lax.broadcasted_iota(jnp.int32, (8, _NL), 1)
            acc = jnp.zeros((8, _NL), jnp.int32)
            for j in range(_MAX_OUT):
                acc = jnp.where((r8 == 0) & (l8 == j), sel_i[j], acc)
                acc = jnp.where((r8 == 1) & (l8 == j), sel_ok[j], acc)
            out_ref[...] = acc


def _gather_body(sref, *refs):
    col_refs = refs[:_C]
    out_ref = refs[_C]
    j = pl.program_id(0)
    l = sref[j] % _NL
    ok = sref[_MAX_OUT + j] > 0
    lane1 = jax.lax.broadcasted_iota(jnp.int32, (1, 1, _NL), 2)
    vals = []
    for c in range(_C):
        row = col_refs[c][...]
        v = jnp.max(jnp.where(lane1 == l, row, _NEG))
        vals.append(jnp.where(c < _C - 1, v * _IMG_SIZE, v))
    row17 = jnp.stack(vals).reshape(1, 1, _C)
    out_ref[...] = jnp.where(ok, row17, 0.0)


@jax.jit
def kernel(detections):
    det = detections.reshape(_N, _C)
    pad = jnp.zeros((_NT - _N,), jnp.float32)

    def col(kk):
        return jnp.concatenate([det[:, kk], pad]).reshape(_NR, _NL)

    cols = [col(kk) for kk in range(_C)]
    cy, cx, hh, ww, sc = cols[0], cols[1], cols[2], cols[3], cols[_C - 1]

    in_spec = pl.BlockSpec(
        (_BR, _NL), lambda k, i: (jnp.where(k == 0, i, 0), 0))
    sel = pl.pallas_call(
        _nms_body,
        out_shape=jax.ShapeDtypeStruct((8, _NL), jnp.int32),
        grid=(_MAX_OUT, _G),
        in_specs=[in_spec] * 5,
        out_specs=pl.BlockSpec((8, _NL), lambda k, i: (0, 0)),
        scratch_shapes=[
            pltpu.VMEM((_NR, _NL), jnp.float32),
            pltpu.VMEM((_NR, _NL), jnp.float32),
            pltpu.VMEM((_NR, _NL), jnp.float32),
            pltpu.VMEM((_NR, _NL), jnp.float32),
            pltpu.VMEM((_NR, _NL), jnp.float32),
            pltpu.SMEM((8,), jnp.float32),
            pltpu.SMEM((8,), jnp.int32),
            pltpu.SMEM((8,), jnp.int32),
            pltpu.SMEM((8,), jnp.int32),
        ],
        compiler_params=pltpu.CompilerParams(
            dimension_semantics=("arbitrary", "arbitrary"),
            vmem_limit_bytes=60_000 * 1024,
        ),
        name="nms_core",
    )(cy, cx, hh, ww, sc)

    idxs = jnp.minimum(sel[0, :_MAX_OUT], _N - 1)
    oks = sel[1, :_MAX_OUT]
    scal = jnp.concatenate([idxs, oks])

    cols3 = [c.reshape(_NR, 1, _NL) for c in cols]
    col_spec = pl.BlockSpec((1, 1, _NL),
                            lambda j, sref: (sref[j] // _NL, 0, 0))
    out3 = pl.pallas_call(
        _gather_body,
        out_shape=jax.ShapeDtypeStruct((_MAX_OUT, 1, _C), jnp.float32),
        grid_spec=pltpu.PrefetchScalarGridSpec(
            num_scalar_prefetch=1,
            grid=(_MAX_OUT,),
            in_specs=[col_spec] * _C,
            out_specs=pl.BlockSpec((1, 1, _C), lambda j, sref: (j, 0, 0)),
        ),
        name="nms_gather",
    )(scal, *cols3)

    return out3.reshape(_MAX_OUT, _C)


# speedup vs baseline: 1.7149x; 1.7149x over previous
import jax
import jax.numpy as jnp
from jax.experimental import pallas as pl
from jax.experimental.pallas import tpu as pltpu

_N = 2_000_000
_C = 17
_MAX_OUT = 5
_IOU_THR = 0.3
_SCORE_THR = 0.5
_IMG_SIZE = 128.0
_CLIP_MAX = 1e8

_NL = 128
_NT = 2_097_152
_NR = _NT // _NL
_BR = 2048
_G = _NR // _BR
_NEG = float("-inf")
_BIG_I = 2 ** 30


def _nms_body(cy_ref, cx_ref, h_ref, w_ref, s_ref, out_ref,
              xs1, ys1, xs2, ys2, sms, smf, smi, sel_i, sel_ok):
    k = pl.program_id(0)
    i = pl.program_id(1)

    rows2d = jax.lax.broadcasted_iota(jnp.int32, (_BR, _NL), 0)
    lanes2d = jax.lax.broadcasted_iota(jnp.int32, (_BR, _NL), 1)

    def update_running(sm):
        bm = jnp.max(sm)
        idxmat = (i * _BR + rows2d) * _NL + lanes2d
        cand = jnp.min(jnp.where(sm == bm, idxmat, _BIG_I))
        cur_m = jnp.where(i == 0, _NEG, smf[0])
        cur_i = jnp.where(i == 0, 0, smi[0])
        better = bm > cur_m
        smf[0] = jnp.where(better, bm, cur_m)
        smi[0] = jnp.where(better, cand, cur_i)

    @pl.when(k == 0)
    def _decode():
        cy = cy_ref[...]
        cx = cx_ref[...]
        h = h_ref[...]
        w = w_ref[...]
        s = s_ref[...]
        y1 = jnp.clip(cy - h * 0.5, 0.0, _CLIP_MAX)
        x1 = jnp.clip(cx - w * 0.5, 0.0, _CLIP_MAX)
        y2 = cy + h * 0.5
        x2 = cx + w * 0.5
        sm = jnp.where(s >= _SCORE_THR, s, _NEG)
        r0 = i * _BR
        xs1[pl.ds(r0, _BR), :] = x1
        ys1[pl.ds(r0, _BR), :] = y1
        xs2[pl.ds(r0, _BR), :] = x2
        ys2[pl.ds(r0, _BR), :] = y2
        sms[pl.ds(r0, _BR), :] = sm
        update_running(sm)

    @pl.when(k > 0)
    def _suppress():
        r0 = i * _BR
        x1 = xs1[pl.ds(r0, _BR), :]
        y1 = ys1[pl.ds(r0, _BR), :]
        x2 = xs2[pl.ds(r0, _BR), :]
        y2 = ys2[pl.ds(r0, _BR), :]
        sm = sms[pl.ds(r0, _BR), :]
        X1 = smf[1]
        Y1 = smf[2]
        X2 = smf[3]
        Y2 = smf[4]
        A = smf[5]
        p_idx = smi[1]
        iw = jnp.maximum(jnp.minimum(x2, X2) - jnp.maximum(x1, X1), 0.0)
        ih = jnp.maximum(jnp.minimum(y2, Y2) - jnp.maximum(y1, Y1), 0.0)
        inter = iw * ih
        areas = (x2 - x1) * (y2 - y1)
        iou = inter / (areas + A - inter + 1e-9)
        idxmat = (i * _BR + rows2d) * _NL + lanes2d
        kill = jnp.logical_or(iou > _IOU_THR, idxmat == p_idx)
        sm = jnp.where(kill, _NEG, sm)
        sms[pl.ds(r0, _BR), :] = sm
        update_running(sm)

    @pl.when(i == _G - 1)
    def _finalize():
        idx = smi[0]
        val = smf[0]
        sel_i[k] = idx
        sel_ok[k] = jnp.where(val > _NEG, 1, 0)
        r = idx // _NL
        l = idx % _NL
        lane1 = jax.lax.broadcasted_iota(jnp.int32, (1, _NL), 1)

        def pick(ref):
            row = ref[pl.ds(r, 1), :]
            return jnp.max(jnp.where(lane1 == l, row, _NEG))

        X1 = pick(xs1)
        Y1 = pick(ys1)
        X2 = pick(xs2)
        Y2 = pick(ys2)
        smf[1] = X1
        smf[2] = Y1
        smf[3] = X2
        smf[4] = Y2
        smf[5] = (X2 - X1) * (Y2 - Y1)
        smi[1] = idx

        @pl.when(k == _MAX_OUT - 1)
        def _emit():
            r8 = jax.lax.broadcasted_iota(jnp.int32, (8, _NL), 0)
            l8 = jax.lax.broadcasted_iota(jnp.int32, (8, _NL), 1)
            acc = jnp.zeros((8, _NL), jnp.int32)
            for j in range(_MAX_OUT):
                acc = jnp.where((r8 == 0) & (l8 == j), sel_i[j], acc)
                acc = jnp.where((r8 == 1) & (l8 == j), sel_ok[j], acc)
            out_ref[...] = acc


@jax.jit
def kernel(detections):
    det = detections.reshape(_N, _C)
    pad = jnp.zeros((_NT - _N,), jnp.float32)

    def col(kk):
        return jnp.concatenate([det[:, kk], pad]).reshape(_NR, _NL)

    cy, cx, hh, ww, sc = col(0), col(1), col(2), col(3), col(_C - 1)

    in_spec = pl.BlockSpec(
        (_BR, _NL), lambda k, i: (jnp.where(k == 0, i, 0), 0))
    sel = pl.pallas_call(
        _nms_body,
        out_shape=jax.ShapeDtypeStruct((8, _NL), jnp.int32),
        grid=(_MAX_OUT, _G),
        in_specs=[in_spec] * 5,
        out_specs=pl.BlockSpec((8, _NL), lambda k, i: (0, 0)),
        scratch_shapes=[
            pltpu.VMEM((_NR, _NL), jnp.float32),
            pltpu.VMEM((_NR, _NL), jnp.float32),
            pltpu.VMEM((_NR, _NL), jnp.float32),
            pltpu.VMEM((_NR, _NL), jnp.float32),
            pltpu.VMEM((_NR, _NL), jnp.float32),
            pltpu.SMEM((8,), jnp.float32),
            pltpu.SMEM((8,), jnp.int32),
            pltpu.SMEM((8,), jnp.int32),
            pltpu.SMEM((8,), jnp.int32),
        ],
        compiler_params=pltpu.CompilerParams(
            dimension_semantics=("arbitrary", "arbitrary"),
            vmem_limit_bytes=60_000 * 1024,
        ),
        name="nms_core",
    )(cy, cx, hh, ww, sc)

    idxs = jnp.minimum(sel[0, :_MAX_OUT], _N - 1)
    oks = sel[1, :_MAX_OUT]
    rows = det[idxs]
    out = jnp.concatenate([rows[:, : _C - 1] * _IMG_SIZE, rows[:, _C - 1:]],
                          axis=1)
    return jnp.where((oks > 0)[:, None], out, 0.0)
